# baseline (speedup 1.0000x reference)
import functools

import jax
import jax.numpy as jnp
from jax import lax
from jax.experimental import pallas as pl
from jax.experimental.pallas import tpu as pltpu
from jax.experimental.pallas import tpu_sc as plsc

DIM = 128
H = 40
HP = 128
NQ = 256
NP = 100000
BP = 16384
NP_PAD = 114688
PAD_DIST = 63
NBINS = 64
K = 64
IDX_BITS = 17
NB16 = NP_PAD // 16
NBLK = NP_PAD // 128
NBLKP = 896
NQH = 128


def _dist_body(q_ref, p_ref, proj_ref, out_ref, min_ref):
    j = pl.program_id(0)
    projT = proj_ref[...].T
    qh = (jnp.dot(q_ref[...], projT,
                  preferred_element_type=jnp.float32) > 0).astype(jnp.float32)
    ph = (jnp.dot(p_ref[...], projT,
                  preferred_element_type=jnp.float32) > 0).astype(jnp.float32)
    sq = jnp.sum(qh, axis=1, keepdims=True)
    sp = jnp.sum(ph, axis=1, keepdims=True).T
    cross = lax.dot_general(qh, ph, (((1,), (1,)), ((), ())),
                            preferred_element_type=jnp.float32)
    dist = sq + sp - 2.0 * cross
    gcol = j * BP + lax.broadcasted_iota(jnp.int32, (1, BP), 1)
    dist = jnp.where(gcol >= NP, float(PAD_DIST), dist)
    t32 = (dist.astype(jnp.int32) << 4) | (gcol & 15)
    out_ref[...] = t32
    min_ref[...] = jnp.min(t32.reshape(NQH, BP // 128, 128),
                           axis=-1).T.reshape(1, BP // 128, NQH)


def _distances(q, p, proj):
    return pl.pallas_call(
        _dist_body,
        grid=(NP_PAD // BP,),
        in_specs=[
            pl.BlockSpec((NQH, DIM), lambda j: (0, 0)),
            pl.BlockSpec((BP, DIM), lambda j: (j, 0)),
            pl.BlockSpec((HP, DIM), lambda j: (0, 0)),
        ],
        out_specs=[
            pl.BlockSpec((NQH, BP), lambda j: (0, j)),
            pl.BlockSpec((1, BP // 128, NQH), lambda j: (j, 0, 0)),
        ],
        out_shape=[
            jax.ShapeDtypeStruct((NQH, NP_PAD), jnp.int32),
            jax.ShapeDtypeStruct((NP_PAD // BP, BP // 128, NQH), jnp.int32),
        ],
    )(q, p, proj)


_MESH = plsc.VectorSubcoreMesh(core_axis_name="c", subcore_axis_name="s")
QPW = NQH // 32
NMV = NBLKP // 16


@functools.partial(
    pl.kernel,
    mesh=_MESH,
    out_type=(
        jax.ShapeDtypeStruct((NQH * 128,), jnp.int32),
        jax.ShapeDtypeStruct((NQH * 128,), jnp.float32),
    ),
    scratch_types=[
        pltpu.VMEM((NBLKP, 128), jnp.int32),
        pltpu.VMEM((QPW, NBLKP), jnp.int32),
        pltpu.VMEM((NBINS * 16,), jnp.int32),
        pltpu.VMEM((NBLKP + 16,), jnp.int32),
        pltpu.VMEM((NBLKP + 16,), jnp.int32),
        pltpu.VMEM((96,), jnp.int32),
        pltpu.VMEM((96,), jnp.int32),
        pltpu.VMEM((QPW * 128 + 32,), jnp.int32),
        pltpu.VMEM((QPW * 128 + 32,), jnp.float32),
        pltpu.SMEM((NBINS,), jnp.int32),
        pltpu.SemaphoreType.DMA,
    ],
    compiler_params=pltpu.CompilerParams(needs_layout_passes=False),
)
def _select(dist_hbm, minv_hbm, oidx_hbm, oval_hbm,
            gbuf, mbuf, hist, cblk, gidx, bufL, bufE, oi, ov, cum, sem):
    wid = lax.axis_index("s") * 2 + lax.axis_index("c")
    iota = lax.iota(jnp.int32, 16)
    ones = jnp.ones((16,), jnp.int32)
    zeros16 = jnp.zeros((16,), jnp.int32)
    lane0 = iota == 0

    def zero_hist(v, carry):
        hist[pl.ds(v * 16, 16)] = zeros16
        return carry

    lax.fori_loop(0, NBINS, zero_hist, 0)
    pltpu.sync_copy(minv_hbm.at[pl.ds(wid * QPW, QPW)], mbuf)

    def per_query(qi, carry):
        q = wid * QPW + qi
        qbase = qi * 128

        @plsc.parallel_loop(0, NMV)
        def s1(i):
            mv = mbuf[qi, pl.ds(i * 16, 16)]
            plsc.addupdate_scatter(hist, [((mv >> 4) << 4) + iota], ones)

        def mscan(v, c):
            run, t = c
            cnt = jnp.sum(hist[pl.ds(v * 16, 16)])
            hist[pl.ds(v * 16, 16)] = zeros16
            run2 = run + cnt
            t = jnp.where((t == NBINS) & (run2 >= K), v, t)
            return run2, t

        _, tmin = lax.fori_loop(0, NBINS, mscan,
                                (jnp.int32(0), jnp.int32(NBINS)))
        t16v = jnp.full((16,), tmin * 16 + 15, jnp.int32)

        def s2(i, nc):
            mv = mbuf[qi, pl.ds(i * 16, 16)]
            msk = mv <= t16v
            plsc.store_compressed(cblk.at[pl.ds(nc, 16)], i * 16 + iota,
                                  mask=msk)
            return nc + jnp.sum(msk.astype(jnp.int32))

        ncb = lax.fori_loop(0, NMV, s2, jnp.int32(0))

        qrow = jnp.full((16,), q * NBLK, jnp.int32)
        ncbv = jnp.full((16,), ncb, jnp.int32)

        @plsc.parallel_loop(0, NMV)
        def build_gidx(i):
            pos = i * 16 + iota
            gid = jnp.where(pos < ncbv, cblk[pl.ds(i * 16, 16)] + qrow, qrow)
            gidx[pl.ds(i * 16, 16)] = gid

        nch = (ncb + 127) >> 7

        def gather(ch, c):
            pltpu.async_copy(dist_hbm.at[gidx.at[pl.ds(ch * 128, 128)]],
                             gbuf.at[pl.ds(ch * 128, 128)], sem).wait()
            return c

        lax.fori_loop(0, nch, gather, 0)

        @plsc.parallel_loop(0, ncb)
        def s4(ci):
            for j in range(8):
                v = gbuf[ci, pl.ds(j * 16, 16)]
                plsc.addupdate_scatter(hist, [v], ones)

        def scan_bins(v, c):
            run, t = c
            cnt = jnp.sum(hist[pl.ds(v * 16, 16)])
            hist[pl.ds(v * 16, 16)] = zeros16
            cum[v] = run
            run2 = run + cnt
            t = jnp.where((t == NBINS) & (run2 >= K), v, t)
            return run2, t

        _, T = lax.fori_loop(0, NBINS, scan_bins,
                             (jnp.int32(0), jnp.int32(NBINS)))
        nL = cum[T]
        eq_target = K - nL
        Tv = jnp.full((16,), T, jnp.int32)

        def b_cond(c):
            ci, offL, offE = c
            return (ci < ncb) & ((offL < nL) | (offE < eq_target))

        def b_body(c):
            ci, offL, offE = c
            b = cblk[pl.ds(ci, 16)][0]
            offs = (ci, offL, offE)
            offL, offE = offs[1], offs[2]
            for j in range(8):
                v = gbuf[ci, pl.ds(j * 16, 16)]
                d = lax.shift_right_logical(v, 4)
                key = d * (1 << IDX_BITS) + (b * 128 + j * 16 + iota)
                mless = d < Tv
                plsc.store_compressed(bufL.at[pl.ds(offL, 16)], key,
                                      mask=mless)
                offL = offL + jnp.sum(mless.astype(jnp.int32))
                open_e = jnp.full((16,), offE < eq_target)
                meq = (d == Tv) & open_e
                plsc.store_compressed(bufE.at[pl.ds(offE, 16)], key,
                                      mask=meq)
                offE = offE + jnp.sum(meq.astype(jnp.int32))
            return ci + 1, offL, offE

        lax.while_loop(b_cond, b_body,
                       (jnp.int32(0), jnp.int32(0), jnp.int32(0)))

        def place_less(jj, c):
            kkey = bufL[pl.ds(jj, 16)][0]
            d = lax.shift_right_logical(kkey, IDX_BITS)
            pos = cum[d]
            cum[d] = pos + 1
            posv = jnp.full((16,), qbase + pos, jnp.int32)
            plsc.store_scatter(oi, [posv],
                               jnp.full((16,), kkey & ((1 << IDX_BITS) - 1),
                                        jnp.int32), mask=lane0)
            plsc.store_scatter(ov, [posv],
                               jnp.full((16,), d.astype(jnp.float32),
                                        jnp.float32), mask=lane0)
            return c

        lax.fori_loop(0, nL, place_less, 0)

        tfv = jnp.full((16,), T.astype(jnp.float32), jnp.float32)

        def place_eq(jj, c):
            kv = bufE[pl.ds(jj * 16, 16)]
            m = (jj * 16 + iota) < eq_target
            plsc.store_compressed(oi.at[pl.ds(qbase + nL + jj * 16, 16)],
                                  kv & ((1 << IDX_BITS) - 1), mask=m)
            plsc.store_compressed(ov.at[pl.ds(qbase + nL + jj * 16, 16)], tfv,
                                  mask=m)
            return c

        lax.fori_loop(0, K // 16, place_eq, 0)
        return carry

    lax.fori_loop(0, QPW, per_query, 0)
    pltpu.sync_copy(oi.at[pl.ds(0, QPW * 128)],
                    oidx_hbm.at[pl.ds(wid * QPW * 128, QPW * 128)])
    pltpu.sync_copy(ov.at[pl.ds(0, QPW * 128)],
                    oval_hbm.at[pl.ds(wid * QPW * 128, QPW * 128)])


def kernel(query_points, points, projection_matrices, k):
    q = query_points[0]
    p = points[0]
    proj = jnp.pad(projection_matrices, ((0, HP - H), (0, 0)))
    idxs, valss = [], []
    for h in range(NQ // NQH):
        dist, minv = _distances(q[h * NQH:(h + 1) * NQH], p, proj)
        minv = minv.reshape(NBLK, NQH).T
        minv = jnp.pad(minv, ((0, 0), (0, NBLKP - NBLK)),
                       constant_values=1023)
        idx, vals = _select(dist.reshape(NQH * NBLK, 128), minv)
        idxs.append(idx.reshape(NQH, 128)[:, :K])
        valss.append(vals.reshape(NQH, 128)[:, :K])
    return (jnp.concatenate(idxs)[None], jnp.concatenate(valss)[None])


# speedup vs baseline: 1.0306x; 1.0306x over previous
import functools

import jax
import jax.numpy as jnp
from jax import lax
from jax.experimental import pallas as pl
from jax.experimental.pallas import tpu as pltpu
from jax.experimental.pallas import tpu_sc as plsc

DIM = 128
H = 40
HP = 128
NQ = 256
NP = 100000
BP = 8192
NP_PAD = 106496
PAD_DIST = 63
NBINS = 64
K = 64
IDX_BITS = 17
NB16 = NP_PAD // 16
NBLK = NP_PAD // 128
NBLKP = 896
NQH = 128


def _dist_body(q_ref, p_ref, proj_ref, out_ref, min_ref):
    j = pl.program_id(0)
    projT = proj_ref[...].T
    qh = (jnp.dot(q_ref[...], projT,
                  preferred_element_type=jnp.float32) > 0).astype(jnp.float32)
    ph = (jnp.dot(p_ref[...], projT,
                  preferred_element_type=jnp.float32) > 0).astype(jnp.float32)
    sq = jnp.sum(qh, axis=1, keepdims=True)
    sp = jnp.sum(ph, axis=1, keepdims=True).T
    cross = lax.dot_general(qh, ph, (((1,), (1,)), ((), ())),
                            preferred_element_type=jnp.float32)
    dist = sq + sp - 2.0 * cross
    gcol = j * BP + lax.broadcasted_iota(jnp.int32, (1, BP), 1)
    dist = jnp.where(gcol >= NP, float(PAD_DIST), dist)
    t32 = (dist.astype(jnp.int32) << 4) | (gcol & 15)
    out_ref[...] = t32
    min_ref[...] = jnp.min(t32.reshape(NQH, BP // 128, 128),
                           axis=-1).T.reshape(1, BP // 128, NQH)


def _distances(q, p, proj):
    return pl.pallas_call(
        _dist_body,
        grid=(NP_PAD // BP,),
        in_specs=[
            pl.BlockSpec((NQH, DIM), lambda j: (0, 0)),
            pl.BlockSpec((BP, DIM), lambda j: (j, 0)),
            pl.BlockSpec((HP, DIM), lambda j: (0, 0)),
        ],
        out_specs=[
            pl.BlockSpec((NQH, BP), lambda j: (0, j)),
            pl.BlockSpec((1, BP // 128, NQH), lambda j: (j, 0, 0)),
        ],
        out_shape=[
            jax.ShapeDtypeStruct((NQH, NP_PAD), jnp.int32),
            jax.ShapeDtypeStruct((NP_PAD // BP, BP // 128, NQH), jnp.int32),
        ],
    )(q, p, proj)


_MESH = plsc.VectorSubcoreMesh(core_axis_name="c", subcore_axis_name="s")
QPW = NQH // 32
NMV = NBLKP // 16


@functools.partial(
    pl.kernel,
    mesh=_MESH,
    out_type=(
        jax.ShapeDtypeStruct((NQH * 128,), jnp.int32),
        jax.ShapeDtypeStruct((NQH * 128,), jnp.float32),
    ),
    scratch_types=[
        pltpu.VMEM((NBLKP, 128), jnp.int32),
        pltpu.VMEM((QPW, NBLKP), jnp.int32),
        pltpu.VMEM((NBINS * 16,), jnp.int32),
        pltpu.VMEM((NBLKP + 16,), jnp.int32),
        pltpu.VMEM((NBLKP + 16,), jnp.int32),
        pltpu.VMEM((96,), jnp.int32),
        pltpu.VMEM((96,), jnp.int32),
        pltpu.VMEM((QPW * 128 + 32,), jnp.int32),
        pltpu.VMEM((QPW * 128 + 32,), jnp.float32),
        pltpu.SMEM((NBINS,), jnp.int32),
        pltpu.SemaphoreType.DMA,
    ],
    compiler_params=pltpu.CompilerParams(needs_layout_passes=False),
)
def _select(dist_hbm, minv_hbm, oidx_hbm, oval_hbm,
            gbuf, mbuf, hist, cblk, gidx, bufL, bufE, oi, ov, cum, sem):
    wid = lax.axis_index("s") * 2 + lax.axis_index("c")
    iota = lax.iota(jnp.int32, 16)
    ones = jnp.ones((16,), jnp.int32)
    zeros16 = jnp.zeros((16,), jnp.int32)
    lane0 = iota == 0

    def zero_hist(v, carry):
        hist[pl.ds(v * 16, 16)] = zeros16
        return carry

    lax.fori_loop(0, NBINS, zero_hist, 0)
    pltpu.sync_copy(minv_hbm.at[pl.ds(wid * QPW, QPW)], mbuf)

    def per_query(qi, carry):
        q = wid * QPW + qi
        qbase = qi * 128

        @plsc.parallel_loop(0, NMV)
        def s1(i):
            mv = mbuf[qi, pl.ds(i * 16, 16)]
            plsc.addupdate_scatter(hist, [((mv >> 4) << 4) + iota], ones)

        def mscan(v, c):
            run, t = c
            cnt = jnp.sum(hist[pl.ds(v * 16, 16)])
            hist[pl.ds(v * 16, 16)] = zeros16
            run2 = run + cnt
            t = jnp.where((t == NBINS) & (run2 >= K), v, t)
            return run2, t

        _, tmin = lax.fori_loop(0, NBINS, mscan,
                                (jnp.int32(0), jnp.int32(NBINS)))
        t16v = jnp.full((16,), tmin * 16 + 15, jnp.int32)

        def s2(i, nc):
            mv = mbuf[qi, pl.ds(i * 16, 16)]
            msk = mv <= t16v
            plsc.store_compressed(cblk.at[pl.ds(nc, 16)], i * 16 + iota,
                                  mask=msk)
            return nc + jnp.sum(msk.astype(jnp.int32))

        ncb = lax.fori_loop(0, NMV, s2, jnp.int32(0))

        qrow = jnp.full((16,), q * NBLK, jnp.int32)
        ncbv = jnp.full((16,), ncb, jnp.int32)

        @plsc.parallel_loop(0, NMV)
        def build_gidx(i):
            pos = i * 16 + iota
            gid = jnp.where(pos < ncbv, cblk[pl.ds(i * 16, 16)] + qrow, qrow)
            gidx[pl.ds(i * 16, 16)] = gid

        nch = (ncb + 127) >> 7

        def gather(ch, c):
            pltpu.async_copy(dist_hbm.at[gidx.at[pl.ds(ch * 128, 128)]],
                             gbuf.at[pl.ds(ch * 128, 128)], sem).wait()
            return c

        lax.fori_loop(0, nch, gather, 0)

        @plsc.parallel_loop(0, ncb)
        def s4(ci):
            for j in range(8):
                v = gbuf[ci, pl.ds(j * 16, 16)]
                plsc.addupdate_scatter(hist, [v], ones)

        def scan_bins(v, c):
            run, t = c
            cnt = jnp.sum(hist[pl.ds(v * 16, 16)])
            hist[pl.ds(v * 16, 16)] = zeros16
            cum[v] = run
            run2 = run + cnt
            t = jnp.where((t == NBINS) & (run2 >= K), v, t)
            return run2, t

        _, T = lax.fori_loop(0, NBINS, scan_bins,
                             (jnp.int32(0), jnp.int32(NBINS)))
        nL = cum[T]
        eq_target = K - nL
        Tv = jnp.full((16,), T, jnp.int32)

        def b_cond(c):
            ci, offL, offE = c
            return (ci < ncb) & ((offL < nL) | (offE < eq_target))

        def b_body(c):
            ci, offL, offE = c
            b = cblk[pl.ds(ci, 16)][0]
            offs = (ci, offL, offE)
            offL, offE = offs[1], offs[2]
            for j in range(8):
                v = gbuf[ci, pl.ds(j * 16, 16)]
                d = lax.shift_right_logical(v, 4)
                key = d * (1 << IDX_BITS) + (b * 128 + j * 16 + iota)
                mless = d < Tv
                plsc.store_compressed(bufL.at[pl.ds(offL, 16)], key,
                                      mask=mless)
                offL = offL + jnp.sum(mless.astype(jnp.int32))
                open_e = jnp.full((16,), offE < eq_target)
                meq = (d == Tv) & open_e
                plsc.store_compressed(bufE.at[pl.ds(offE, 16)], key,
                                      mask=meq)
                offE = offE + jnp.sum(meq.astype(jnp.int32))
            return ci + 1, offL, offE

        lax.while_loop(b_cond, b_body,
                       (jnp.int32(0), jnp.int32(0), jnp.int32(0)))

        def place_less(jj, c):
            kkey = bufL[pl.ds(jj, 16)][0]
            d = lax.shift_right_logical(kkey, IDX_BITS)
            pos = cum[d]
            cum[d] = pos + 1
            posv = jnp.full((16,), qbase + pos, jnp.int32)
            plsc.store_scatter(oi, [posv],
                               jnp.full((16,), kkey & ((1 << IDX_BITS) - 1),
                                        jnp.int32), mask=lane0)
            plsc.store_scatter(ov, [posv],
                               jnp.full((16,), d.astype(jnp.float32),
                                        jnp.float32), mask=lane0)
            return c

        lax.fori_loop(0, nL, place_less, 0)

        tfv = jnp.full((16,), T.astype(jnp.float32), jnp.float32)

        def place_eq(jj, c):
            kv = bufE[pl.ds(jj * 16, 16)]
            m = (jj * 16 + iota) < eq_target
            plsc.store_compressed(oi.at[pl.ds(qbase + nL + jj * 16, 16)],
                                  kv & ((1 << IDX_BITS) - 1), mask=m)
            plsc.store_compressed(ov.at[pl.ds(qbase + nL + jj * 16, 16)], tfv,
                                  mask=m)
            return c

        lax.fori_loop(0, K // 16, place_eq, 0)
        return carry

    lax.fori_loop(0, QPW, per_query, 0)
    pltpu.sync_copy(oi.at[pl.ds(0, QPW * 128)],
                    oidx_hbm.at[pl.ds(wid * QPW * 128, QPW * 128)])
    pltpu.sync_copy(ov.at[pl.ds(0, QPW * 128)],
                    oval_hbm.at[pl.ds(wid * QPW * 128, QPW * 128)])


def kernel(query_points, points, projection_matrices, k):
    q = query_points[0]
    p = points[0]
    proj = jnp.pad(projection_matrices, ((0, HP - H), (0, 0)))
    idxs, valss = [], []
    for h in range(NQ // NQH):
        dist, minv = _distances(q[h * NQH:(h + 1) * NQH], p, proj)
        minv = minv.reshape(NBLK, NQH).T
        minv = jnp.pad(minv, ((0, 0), (0, NBLKP - NBLK)),
                       constant_values=1023)
        idx, vals = _select(dist.reshape(NQH * NBLK, 128), minv)
        idxs.append(idx.reshape(NQH, 128)[:, :K])
        valss.append(vals.reshape(NQH, 128)[:, :K])
    return (jnp.concatenate(idxs)[None], jnp.concatenate(valss)[None])
